# baseline (speedup 1.0000x reference)
import functools

import jax
import jax.numpy as jnp
from jax import lax
from jax.experimental import pallas as pl
from jax.experimental.pallas import tpu as pltpu
from jax.experimental.pallas import tpu_sc as plsc

_L = 16


def _ceil_div(a, b):
  return -(-a // b)


@functools.lru_cache(maxsize=None)
def _make_sc_aggregate(n_nodes, d_feat, n_edges):
  NC, NS = 2, 16
  NW = NC * NS
  ED = 128
  GROUP = 8
  steps = _ceil_div(n_edges, NW * ED * GROUP) * GROUP
  pair_steps = steps * NC
  s0 = int(round(pair_steps * 0.65 / GROUP)) * GROUP
  s1 = pair_steps - s0
  npad = _ceil_div(n_nodes + 1, NS * ED) * NS * ED
  rpt = npad // NS

  mesh = plsc.VectorSubcoreMesh(core_axis_name="c", subcore_axis_name="s")

  @functools.partial(
      pl.kernel,
      out_type=(
          jax.ShapeDtypeStruct((NC, npad, d_feat), jnp.float32),
          jax.ShapeDtypeStruct((NC, npad, d_feat), jnp.float32),
      ),
      mesh=mesh,
      scratch_types=[
          pltpu.VMEM((GROUP, ED), jnp.int32),
          pltpu.VMEM((GROUP, ED), jnp.int32),
          pltpu.VMEM((ED, d_feat), jnp.float32),
          pltpu.VMEM((ED, d_feat), jnp.float32),
          pltpu.VMEM_SHARED((npad, d_feat), jnp.float32),
          pltpu.SemaphoreType.DMA,
          pltpu.SemaphoreType.DMA,
          pltpu.SemaphoreType.DMA,
          pltpu.SemaphoreType.DMA,
      ],
  )
  def sc_aggregate(src_hbm, dst_hbm, x_hbm, zrow_hbm,
                   agg_out, cnt_out, src_v, dst_v, rows_a, rows_b, agg_sh,
                   sem_ga, sem_gb, sem_sa, sem_sb):
    cid = lax.axis_index("c")
    sid = lax.axis_index("s")
    base_c = cid * s0
    steps_c = jnp.where(cid == 0, s0, s1)
    bufs = (rows_a, rows_b)
    gsems = (sem_ga, sem_gb)
    ssems = (sem_sa, sem_sb)

    pltpu.sync_copy(
        zrow_hbm.at[pl.ds(sid * rpt, rpt)],
        agg_sh.at[pl.ds(sid * rpt, rpt)])
    plsc.subcore_barrier()

    @pl.loop(0, steps_c // GROUP)
    def _(g):
      off = pl.multiple_of(g * GROUP + base_c, GROUP)
      pltpu.sync_copy(src_hbm.at[sid].at[pl.ds(off, GROUP)], src_v)
      pltpu.sync_copy(dst_hbm.at[sid].at[pl.ds(off, GROUP)], dst_v)
      dg = {0: pltpu.async_copy(x_hbm.at[src_v.at[0]], rows_a, sem_ga)}
      ds = {}
      for j in range(GROUP):
        b = j % 2
        nb = 1 - b
        if j + 1 < GROUP:
          if j >= 1:
            ds[j - 1].wait()
          dg[j + 1] = pltpu.async_copy(
              x_hbm.at[src_v.at[j + 1]], bufs[nb], gsems[nb])
        dg[j].wait()
        ds[j] = pltpu.async_copy(
            bufs[b], agg_sh.at[dst_v.at[j]], ssems[b], add=True)
      ds[GROUP - 2].wait()
      ds[GROUP - 1].wait()

    plsc.subcore_barrier()
    pltpu.sync_copy(
        agg_sh.at[pl.ds(sid * rpt, rpt)],
        agg_out.at[cid].at[pl.ds(sid * rpt, rpt)])

    pltpu.sync_copy(
        zrow_hbm.at[pl.ds(sid * rpt, rpt)],
        agg_sh.at[pl.ds(sid * rpt, rpt)])

    @pl.loop(0, ED)
    def _(i):
      for m in range(d_feat // _L):
        rows_a[i, pl.ds(m * _L, _L)] = jnp.full((_L,), 1.0, jnp.float32)

    plsc.subcore_barrier()

    @pl.loop(0, steps_c // GROUP)
    def _(g):
      off = pl.multiple_of(g * GROUP + base_c, GROUP)
      pltpu.sync_copy(dst_hbm.at[sid].at[pl.ds(off, GROUP)], dst_v)
      ds = [
          pltpu.async_copy(rows_a, agg_sh.at[dst_v.at[j]], sem_sa, add=True)
          for j in range(GROUP)
      ]
      for d in ds:
        d.wait()

    plsc.subcore_barrier()
    pltpu.sync_copy(
        agg_sh.at[pl.ds(sid * rpt, rpt)],
        cnt_out.at[cid].at[pl.ds(sid * rpt, rpt)])

  return sc_aggregate, steps, ED, NW, npad


def _tc_forward_body(aggp_ref, cntp_ref, x_ref, wl_ref, bl_ref, wr_ref,
                     wlin_ref, blin_ref, out_ref):
  agg = aggp_ref[0] + aggp_ref[1]
  cnt = cntp_ref[0, :, 0:1] + cntp_ref[1, :, 0:1]
  inv = 1.0 / jnp.maximum(cnt, 1.0)
  mean = agg * inv
  h = jnp.dot(mean, wl_ref[...], preferred_element_type=jnp.float32)
  h = h + bl_ref[...]
  h = h + jnp.dot(x_ref[...], wr_ref[...], preferred_element_type=jnp.float32)
  h = jnp.maximum(h, 0.0)
  out_ref[...] = (
      jnp.dot(h, wlin_ref[...], preferred_element_type=jnp.float32)
      + blin_ref[...])


def kernel(x_comment, edge_index, W_l, b_l, W_r, W_lin, b_lin):
  n, d = x_comment.shape
  e = edge_index.shape[1]
  hidden = W_l.shape[1]
  out_dim = W_lin.shape[1]

  ei = edge_index.astype(jnp.int32)
  ei = jnp.where(ei > n - 1, 0, ei)
  src, dst = ei[0], ei[1]

  sc_aggregate, steps, ed, nw, npad = _make_sc_aggregate(n, d, e)
  ep = nw * steps * ed
  ns = nw // 2
  src_p = jnp.concatenate(
      [src, jnp.zeros((ep - e,), jnp.int32)]).reshape(ns, 2 * steps, ed)
  dst_p = jnp.concatenate(
      [dst, jnp.full((ep - e,), n, jnp.int32)]).reshape(ns, 2 * steps, ed)
  zrow = jnp.zeros((npad, d), jnp.float32)

  aggp, cntp = sc_aggregate(src_p, dst_p, x_comment, zrow)

  rows = 2000
  grid = n // rows
  out = pl.pallas_call(
      _tc_forward_body,
      grid=(grid,),
      in_specs=[
          pl.BlockSpec((2, rows, d), lambda i: (0, i, 0)),
          pl.BlockSpec((2, rows, d), lambda i: (0, i, 0)),
          pl.BlockSpec((rows, d), lambda i: (i, 0)),
          pl.BlockSpec((d, hidden), lambda i: (0, 0)),
          pl.BlockSpec((1, hidden), lambda i: (0, 0)),
          pl.BlockSpec((d, hidden), lambda i: (0, 0)),
          pl.BlockSpec((hidden, out_dim), lambda i: (0, 0)),
          pl.BlockSpec((1, out_dim), lambda i: (0, 0)),
      ],
      out_specs=pl.BlockSpec((rows, out_dim), lambda i: (i, 0)),
      out_shape=jax.ShapeDtypeStruct((n, out_dim), jnp.float32),
  )(aggp, cntp, x_comment, W_l, b_l.reshape(1, -1), W_r, W_lin,
    b_lin.reshape(1, -1))
  return out


# speedup vs baseline: 1.0832x; 1.0832x over previous
import functools

import jax
import jax.numpy as jnp
from jax import lax
from jax.experimental import pallas as pl
from jax.experimental.pallas import tpu as pltpu
from jax.experimental.pallas import tpu_sc as plsc

_L = 16


def _ceil_div(a, b):
  return -(-a // b)


@functools.lru_cache(maxsize=None)
def _make_sc_aggregate(n_nodes, d_feat, n_edges):
  NC, NS = 2, 16
  NW = NC * NS
  ED = 128
  GROUP = 8
  steps = _ceil_div(n_edges, NW * ED * GROUP) * GROUP
  pair_steps = steps * NC
  s0 = int(round(pair_steps * 0.7 / GROUP)) * GROUP
  s1 = pair_steps - s0
  npad = _ceil_div(n_nodes + 1, NS * ED) * NS * ED
  rpt = npad // NS

  mesh = plsc.VectorSubcoreMesh(core_axis_name="c", subcore_axis_name="s")

  @functools.partial(
      pl.kernel,
      out_type=(
          jax.ShapeDtypeStruct((NC, npad, d_feat), jnp.float32),
          jax.ShapeDtypeStruct((NC, npad, d_feat), jnp.float32),
      ),
      mesh=mesh,
      scratch_types=[
          pltpu.VMEM((2, GROUP, ED), jnp.int32),
          pltpu.VMEM((2, GROUP, ED), jnp.int32),
          pltpu.VMEM((ED, d_feat), jnp.float32),
          pltpu.VMEM((ED, d_feat), jnp.float32),
          pltpu.VMEM_SHARED((npad, d_feat), jnp.float32),
          pltpu.SemaphoreType.DMA,
          pltpu.SemaphoreType.DMA,
          pltpu.SemaphoreType.DMA,
          pltpu.SemaphoreType.DMA,
          pltpu.SemaphoreType.DMA,
          pltpu.SemaphoreType.DMA,
      ],
  )
  def sc_aggregate(src_hbm, dst_hbm, x_hbm, zrow_hbm,
                   agg_out, cnt_out, src_v, dst_v, rows_a, rows_b, agg_sh,
                   sem_ga, sem_gb, sem_sa, sem_sb, sem_ia, sem_ib):
    cid = lax.axis_index("c")
    sid = lax.axis_index("s")
    base_c = cid * s0
    steps_c = jnp.where(cid == 0, s0, s1)
    n_groups = steps_c // GROUP
    bufs = (rows_a, rows_b)
    gsems = (sem_ga, sem_gb)
    ssems = (sem_sa, sem_sb)
    isems = (sem_ia, sem_ib)

    def stage_idx(g, slot, what):
      @pl.when(g < n_groups)
      def _():
        off = pl.multiple_of(g * GROUP + base_c, GROUP)
        if what in ("src", "both"):
          pltpu.async_copy(
              src_hbm.at[sid].at[pl.ds(off, GROUP)], src_v.at[slot],
              isems[slot])
        if what in ("dst", "both"):
          pltpu.async_copy(
              dst_hbm.at[sid].at[pl.ds(off, GROUP)], dst_v.at[slot],
              isems[slot])

    def wait_idx(slot, n_copies):
      for _ in range(n_copies):
        pltpu.make_async_copy(
            dst_hbm.at[sid].at[pl.ds(0, GROUP)], dst_v.at[slot],
            isems[slot]).wait()

    def run_group(slot):
      dg = {0: pltpu.async_copy(
          x_hbm.at[src_v.at[slot].at[0]], rows_a, sem_ga)}
      ds = {}
      for j in range(GROUP):
        b = j % 2
        nb = 1 - b
        if j + 1 < GROUP:
          if j >= 1:
            ds[j - 1].wait()
          dg[j + 1] = pltpu.async_copy(
              x_hbm.at[src_v.at[slot].at[j + 1]], bufs[nb], gsems[nb])
        dg[j].wait()
        ds[j] = pltpu.async_copy(
            bufs[b], agg_sh.at[dst_v.at[slot].at[j]], ssems[b], add=True)
      ds[GROUP - 2].wait()
      ds[GROUP - 1].wait()

    pltpu.sync_copy(
        zrow_hbm.at[pl.ds(sid * rpt, rpt)],
        agg_sh.at[pl.ds(sid * rpt, rpt)])
    plsc.subcore_barrier()

    stage_idx(0, 0, "both")
    stage_idx(1, 1, "both")

    @pl.loop(0, (steps_c // GROUP + 1) // 2)
    def _(q):
      g = q * 2
      wait_idx(0, 2)
      run_group(0)
      stage_idx(g + 2, 0, "both")

      @pl.when(g + 1 < n_groups)
      def _():
        wait_idx(1, 2)
        run_group(1)
        stage_idx(g + 3, 1, "both")

    plsc.subcore_barrier()
    pltpu.sync_copy(
        agg_sh.at[pl.ds(sid * rpt, rpt)],
        agg_out.at[cid].at[pl.ds(sid * rpt, rpt)])

    pltpu.sync_copy(
        zrow_hbm.at[pl.ds(sid * rpt, rpt)],
        agg_sh.at[pl.ds(sid * rpt, rpt)])

    @pl.loop(0, ED)
    def _(i):
      for m in range(d_feat // _L):
        rows_a[i, pl.ds(m * _L, _L)] = jnp.full((_L,), 1.0, jnp.float32)

    plsc.subcore_barrier()

    def fire_group(slot):
      for j in range(GROUP):
        pltpu.async_copy(
            rows_a, agg_sh.at[dst_v.at[slot].at[j]], ssems[slot], add=True)

    def drain_group(slot):
      for _ in range(GROUP):
        pltpu.make_async_copy(
            rows_a, agg_sh.at[dst_v.at[slot].at[0]], ssems[slot]).wait()

    stage_idx(0, 0, "dst")
    stage_idx(1, 1, "dst")

    @pl.loop(0, (steps_c // GROUP + 1) // 2)
    def _(q):
      g = q * 2
      wait_idx(0, 1)
      fire_group(0)

      @pl.when(g + 1 < n_groups)
      def _():
        wait_idx(1, 1)
        fire_group(1)

      drain_group(0)
      stage_idx(g + 2, 0, "dst")

      @pl.when(g + 1 < n_groups)
      def _():
        drain_group(1)
        stage_idx(g + 3, 1, "dst")

    plsc.subcore_barrier()
    pltpu.sync_copy(
        agg_sh.at[pl.ds(sid * rpt, rpt)],
        cnt_out.at[cid].at[pl.ds(sid * rpt, rpt)])

  return sc_aggregate, steps, ED, NW, npad


def _tc_forward_body(aggp_ref, cntp_ref, x_ref, wl_ref, bl_ref, wr_ref,
                     wlin_ref, blin_ref, out_ref):
  agg = aggp_ref[0] + aggp_ref[1]
  cnt = cntp_ref[0, :, 0:1] + cntp_ref[1, :, 0:1]
  inv = 1.0 / jnp.maximum(cnt, 1.0)
  mean = agg * inv
  h = jnp.dot(mean, wl_ref[...], preferred_element_type=jnp.float32)
  h = h + bl_ref[...]
  h = h + jnp.dot(x_ref[...], wr_ref[...], preferred_element_type=jnp.float32)
  h = jnp.maximum(h, 0.0)
  out_ref[...] = (
      jnp.dot(h, wlin_ref[...], preferred_element_type=jnp.float32)
      + blin_ref[...])


def kernel(x_comment, edge_index, W_l, b_l, W_r, W_lin, b_lin):
  n, d = x_comment.shape
  e = edge_index.shape[1]
  hidden = W_l.shape[1]
  out_dim = W_lin.shape[1]

  ei = edge_index.astype(jnp.int32)
  ei = jnp.where(ei > n - 1, 0, ei)
  src, dst = ei[0], ei[1]

  sc_aggregate, steps, ed, nw, npad = _make_sc_aggregate(n, d, e)
  ep = nw * steps * ed
  ns = nw // 2
  src_p = jnp.concatenate(
      [src, jnp.zeros((ep - e,), jnp.int32)]).reshape(ns, 2 * steps, ed)
  dst_p = jnp.concatenate(
      [dst, jnp.full((ep - e,), n, jnp.int32)]).reshape(ns, 2 * steps, ed)
  zrow = jnp.zeros((npad, d), jnp.float32)

  aggp, cntp = sc_aggregate(src_p, dst_p, x_comment, zrow)

  rows = 2000
  grid = n // rows
  out = pl.pallas_call(
      _tc_forward_body,
      grid=(grid,),
      in_specs=[
          pl.BlockSpec((2, rows, d), lambda i: (0, i, 0)),
          pl.BlockSpec((2, rows, d), lambda i: (0, i, 0)),
          pl.BlockSpec((rows, d), lambda i: (i, 0)),
          pl.BlockSpec((d, hidden), lambda i: (0, 0)),
          pl.BlockSpec((1, hidden), lambda i: (0, 0)),
          pl.BlockSpec((d, hidden), lambda i: (0, 0)),
          pl.BlockSpec((hidden, out_dim), lambda i: (0, 0)),
          pl.BlockSpec((1, out_dim), lambda i: (0, 0)),
      ],
      out_specs=pl.BlockSpec((rows, out_dim), lambda i: (i, 0)),
      out_shape=jax.ShapeDtypeStruct((n, out_dim), jnp.float32),
  )(aggp, cntp, x_comment, W_l, b_l.reshape(1, -1), W_r, W_lin,
    b_lin.reshape(1, -1))
  return out
